# Initial kernel scaffold; baseline (speedup 1.0000x reference)
import jax
import jax.numpy as jnp
from jax.experimental import pallas as pl


def kernel(x, edge_index, edge_values):
    raise NotImplementedError("write your pallas kernel here")


# speedup vs baseline: 2.0407x; 2.0407x over previous
import functools

import jax
import jax.numpy as jnp
from jax import lax
from jax.experimental import pallas as pl
from jax.experimental.pallas import tpu as pltpu
from jax.experimental.pallas import tpu_sc as plsc

N_NODES = 10000
N_EDGES = 320000
D_FEAT = 128
D_HALF = D_FEAT // 2

NUM_SUBCORES = 16
CHUNK = 128
CHUNKS_PER_TILE = 160
EDGES_PER_TILE = CHUNK * CHUNKS_PER_TILE
N_EDGES_PAD = EDGES_PER_TILE * NUM_SUBCORES
N_NODES_PAD = 10240
ROWS_PER_TILE = N_NODES_PAD // NUM_SUBCORES

_mesh = plsc.VectorSubcoreMesh(core_axis_name="c", subcore_axis_name="s")


@functools.partial(
    pl.kernel,
    out_type=jax.ShapeDtypeStruct((2, N_NODES_PAD, D_HALF), jnp.float32),
    mesh=_mesh,
    compiler_params=pltpu.CompilerParams(use_tc_tiling_on_sc=False),
    scratch_types=[
        pltpu.VMEM((CHUNK,), jnp.int32),
        pltpu.VMEM((CHUNK,), jnp.int32),
        pltpu.VMEM((CHUNK,), jnp.float32),
        pltpu.VMEM((CHUNK, D_HALF), jnp.float32),
        pltpu.VMEM((ROWS_PER_TILE, D_HALF), jnp.float32),
        pltpu.VMEM_SHARED((N_NODES_PAD, D_HALF), jnp.float32),
        pltpu.SemaphoreType.DMA,
    ],
)
def _spmm_sc(x2_h, col2_h, row_h, val_h, out_h,
             idx_v, row_v, val_v, buf, stage, acc, sem):
    c = lax.axis_index("c")
    s = lax.axis_index("s")

    def zero_body(i, carry):
        for f in range(D_HALF // 16):
            stage[i, pl.ds(f * 16, 16)] = jnp.zeros((16,), jnp.float32)
        return carry
    lax.fori_loop(0, ROWS_PER_TILE, zero_body, 0)
    pltpu.sync_copy(stage, acc.at[pl.ds(s * ROWS_PER_TILE, ROWS_PER_TILE)])
    plsc.subcore_barrier()

    base0 = s * EDGES_PER_TILE

    def chunk_body(k, carry):
        base = base0 + k * CHUNK
        pltpu.sync_copy(col2_h.at[c, pl.ds(base, CHUNK)], idx_v)
        pltpu.sync_copy(row_h.at[pl.ds(base, CHUNK)], row_v)
        pltpu.sync_copy(val_h.at[pl.ds(base, CHUNK)], val_v)
        pltpu.async_copy(x2_h.at[idx_v], buf, sem).wait()

        def scale_body(g, inner):
            e0 = g * 16
            vv = val_v[pl.ds(e0, 16)]
            for j in range(16):
                v = vv[j]
                for f in range(D_HALF // 16):
                    sl = pl.ds(f * 16, 16)
                    buf[e0 + j, sl] = buf[e0 + j, sl] * v
            return inner
        lax.fori_loop(0, CHUNK // 16, scale_body, 0)

        pltpu.sync_copy(buf, acc.at[row_v], add=True)
        return carry

    lax.fori_loop(0, CHUNKS_PER_TILE, chunk_body, 0)
    plsc.subcore_barrier()

    slab = pl.ds(s * ROWS_PER_TILE, ROWS_PER_TILE)
    pltpu.sync_copy(acc.at[slab], stage)
    pltpu.sync_copy(stage, out_h.at[c, slab])


def kernel(x, edge_index, edge_values):
    row = edge_index[0].astype(jnp.int32)
    col = edge_index[1].astype(jnp.int32)
    vals = edge_values.astype(jnp.float32)
    pad = N_EDGES_PAD - N_EDGES
    row_p = jnp.pad(row, (0, pad))
    col_p = jnp.pad(col, (0, pad))
    val_p = jnp.pad(vals, (0, pad))
    col2 = jnp.stack([col_p, col_p + N_NODES])
    x2 = jnp.concatenate([x[:, :D_HALF], x[:, D_HALF:]], axis=0)
    out = _spmm_sc(x2, col2, row_p, val_p)
    return jnp.concatenate([out[0, :N_NODES], out[1, :N_NODES]], axis=1)


# speedup vs baseline: 5.5309x; 2.7103x over previous
import functools

import jax
import jax.numpy as jnp
from jax import lax
from jax.experimental import pallas as pl
from jax.experimental.pallas import tpu as pltpu
from jax.experimental.pallas import tpu_sc as plsc

N_NODES = 10000
N_EDGES = 320000
D_FEAT = 128
D_HALF = D_FEAT // 2

NUM_SUBCORES = 16
CHUNK = 128
CHUNKS_PER_TILE = 160
EDGES_PER_TILE = CHUNK * CHUNKS_PER_TILE
N_EDGES_PAD = EDGES_PER_TILE * NUM_SUBCORES
N_NODES_PAD = 10240
ROWS_PER_TILE = N_NODES_PAD // NUM_SUBCORES
NBUF = 2
PHASES = 2
PCH = CHUNKS_PER_TILE // PHASES
PEDGES = PCH * CHUNK

_mesh = plsc.VectorSubcoreMesh(core_axis_name="c", subcore_axis_name="s")


@functools.partial(
    pl.kernel,
    out_type=jax.ShapeDtypeStruct((2, N_NODES_PAD, D_HALF), jnp.float32),
    mesh=_mesh,
    compiler_params=pltpu.CompilerParams(use_tc_tiling_on_sc=False),
    scratch_types=[
        pltpu.VMEM((PEDGES,), jnp.int32),
        pltpu.VMEM((PCH, CHUNK), jnp.int32),
        pltpu.VMEM((PEDGES,), jnp.float32),
        pltpu.VMEM((CHUNK, D_HALF), jnp.float32),
        pltpu.VMEM((CHUNK, D_HALF), jnp.float32),
        pltpu.VMEM((CHUNK, D_HALF), jnp.float32),
        pltpu.VMEM((CHUNK, D_HALF), jnp.float32),
        pltpu.VMEM((CHUNK,), jnp.int32),
        pltpu.VMEM((CHUNK,), jnp.int32),
        pltpu.VMEM_SHARED((N_NODES_PAD, D_HALF), jnp.float32),
        pltpu.SemaphoreType.DMA,
        pltpu.SemaphoreType.DMA,
        pltpu.SemaphoreType.DMA,
        pltpu.SemaphoreType.DMA,
    ],
)
def _spmm_sc(x2_h, col2_h, row3_h, val_h, out_h,
             colv, rowv, valv, gbuf0, gbuf1, sbuf0, sbuf1, ridx0, ridx1, acc,
             gsem0, gsem1, ssem0, ssem1):
    c = lax.axis_index("c")
    s = lax.axis_index("s")
    gbuf = (gbuf0, gbuf1)
    sbuf = (sbuf0, sbuf1)
    ridx = (ridx0, ridx1)
    gsem = (gsem0, gsem1)
    ssem = (ssem0, ssem1)

    def zero_body(i, carry):
        for f in range(D_HALF // 16):
            gbuf0[i, pl.ds(f * 16, 16)] = jnp.zeros((16,), jnp.float32)
        return carry
    lax.fori_loop(0, CHUNK, zero_body, 0)
    for i in range(ROWS_PER_TILE // CHUNK):
        pltpu.sync_copy(gbuf0, acc.at[pl.ds(s * ROWS_PER_TILE + i * CHUNK,
                                            CHUNK)])
    plsc.subcore_barrier()

    def gather_desc(q, b):
        idx = colv.at[pl.ds(q * CHUNK, CHUNK)]
        return pltpu.make_async_copy(x2_h.at[idx], gbuf[b], gsem[b])

    def scatter_start(b):
        pltpu.async_copy(sbuf[b], acc.at[ridx[b]], ssem[b], add=True)

    def scatter_wait(b):
        pltpu.make_async_copy(sbuf[b], acc.at[ridx[b]], ssem[b]).wait()

    for h in range(PHASES):
        e_lo = s * EDGES_PER_TILE + h * PEDGES
        pltpu.sync_copy(col2_h.at[c, pl.ds(e_lo, PEDGES)], colv)
        pltpu.sync_copy(row3_h.at[s, pl.ds(h * PCH, PCH)], rowv)
        pltpu.sync_copy(val_h.at[pl.ds(e_lo, PEDGES)], valv)

        for b in range(NBUF):
            gather_desc(b, b).start()

        def outer_body(o, carry):
            for b in range(NBUF):
                q = o * NBUF + b
                gather_desc(q, b).wait()

                @pl.when(q >= NBUF)
                def _():
                    scatter_wait(b)

                for g in range(CHUNK // 16):
                    sl = pl.ds(g * 16, 16)
                    ridx[b][sl] = rowv[q, sl]

                for g in range(CHUNK // 16):
                    e0 = g * 16
                    vv = valv[pl.ds(q * CHUNK + e0, 16)]
                    for j in range(16):
                        v = vv[j]
                        for f in range(D_HALF // 16):
                            sl = pl.ds(f * 16, 16)
                            sbuf[b][e0 + j, sl] = gbuf[b][e0 + j, sl] * v

                scatter_start(b)

                qn = q + NBUF

                @pl.when(qn < PCH)
                def _():
                    gather_desc(qn, b).start()
            return carry

        lax.fori_loop(0, PCH // NBUF, outer_body, 0)

        for b in range(NBUF):
            scatter_wait(b)

    plsc.subcore_barrier()

    for i in range(ROWS_PER_TILE // CHUNK):
        sl = pl.ds(s * ROWS_PER_TILE + i * CHUNK, CHUNK)
        pltpu.sync_copy(acc.at[sl], gbuf0)
        pltpu.sync_copy(gbuf0, out_h.at[c, sl])


def kernel(x, edge_index, edge_values):
    row = edge_index[0].astype(jnp.int32)
    col = edge_index[1].astype(jnp.int32)
    vals = edge_values.astype(jnp.float32)
    pad = N_EDGES_PAD - N_EDGES
    row_p = jnp.pad(row, (0, pad)).reshape(NUM_SUBCORES, CHUNKS_PER_TILE,
                                           CHUNK)
    col_p = jnp.pad(col, (0, pad))
    val_p = jnp.pad(vals, (0, pad))
    col2 = jnp.stack([col_p, col_p + N_NODES])
    x2 = jnp.concatenate([x[:, :D_HALF], x[:, D_HALF:]], axis=0)
    out = _spmm_sc(x2, col2, row_p, val_p)
    return jnp.concatenate([out[0, :N_NODES], out[1, :N_NODES]], axis=1)
